# Initial kernel scaffold; baseline (speedup 1.0000x reference)
import jax
import jax.numpy as jnp
from jax.experimental import pallas as pl


def kernel(x, u_size, v_size):
    raise NotImplementedError("write your pallas kernel here")


# speedup vs baseline: 6.2009x; 6.2009x over previous
import functools

import jax
import jax.numpy as jnp
from jax import lax
from jax.experimental import pallas as pl
from jax.experimental.pallas import tpu as pltpu
from jax.experimental.pallas import tpu_sc as plsc


def _greedy_sc(x):
    B, V, U = x.shape
    info = plsc.get_sparse_core_info()
    NC, NS, L = info.num_cores, info.num_subcores, info.num_lanes
    NW = NC * NS
    IPW = B // NW
    NREG = (U + L - 1) // L
    mesh = plsc.VectorSubcoreMesh(core_axis_name="c", subcore_axis_name="s")

    @functools.partial(
        pl.kernel,
        out_type=(
            jax.ShapeDtypeStruct((B,), jnp.float32),
            jax.ShapeDtypeStruct((B, V), jnp.int32),
        ),
        mesh=mesh,
        scratch_types=[
            pltpu.VMEM((V, U), jnp.float32),
            pltpu.VMEM((V,), jnp.int32),
            pltpu.VMEM((IPW,), jnp.float32),
        ],
        compiler_params=pltpu.CompilerParams(needs_layout_passes=False),
    )
    def greedy(x_hbm, size_hbm, seq_hbm, wbuf, seqrow, sizebuf):
        wid = lax.axis_index("s") * NC + lax.axis_index("c")
        base = wid * IPW
        iota = lax.iota(jnp.int32, L)
        lane0 = iota == 0
        offs = [j * L for j in range(NREG - 1)] + [U - L]
        idxs = [iota + o for o in offs]

        @pl.loop(0, IPW)
        def inst_loop(i):
            b = base + i
            pltpu.sync_copy(x_hbm.at[b], wbuf)

            def step(t, carry):
                size_vec = carry[0]
                ms = carry[1:]
                ws = [wbuf[t, pl.ds(o, L)] + m for o, m in zip(offs, ms)]
                mx = ws[0]
                for wv in ws[1:]:
                    mx = jnp.maximum(mx, wv)
                gmax = lax.reduce_max(mx, (0,))
                gmax_vec = jnp.full((L,), gmax, dtype=jnp.float32)
                cand = [
                    jnp.where(wv == gmax_vec, iv, jnp.int32(4 * L * NREG))
                    for wv, iv in zip(ws, idxs)
                ]
                mn = cand[0]
                for cv in cand[1:]:
                    mn = jnp.minimum(mn, cv)
                sel = lax.reduce_min(mn, (0,))
                sel_vec = jnp.full((L,), sel, dtype=jnp.int32)
                plsc.store_scatter(
                    seqrow,
                    [jnp.full((L,), t, dtype=jnp.int32)],
                    sel_vec,
                    mask=lane0,
                )
                nz = sel_vec != 0
                size_vec = size_vec - jnp.where(nz, gmax_vec, 0.0)
                sel_upd = jnp.where(nz, sel_vec, jnp.int32(-1))
                ms = tuple(
                    jnp.where(iv == sel_upd, jnp.float32(-2.0), m)
                    for iv, m in zip(idxs, ms)
                )
                return (size_vec,) + ms

            zero_v = jnp.zeros((L,), jnp.float32)
            carry = (zero_v,) + tuple(zero_v for _ in range(NREG))
            out_carry = lax.fori_loop(0, V, step, carry)
            size_vec = out_carry[0]
            pltpu.sync_copy(seqrow, seq_hbm.at[b])
            plsc.store_scatter(
                sizebuf,
                [jnp.full((L,), i, dtype=jnp.int32)],
                size_vec,
                mask=lane0,
            )

        pltpu.sync_copy(sizebuf, size_hbm.at[pl.ds(base, IPW)])

    return greedy(x)


def kernel(x, u_size, v_size):
    del u_size, v_size
    neg_size, seqs = _greedy_sc(x)
    return neg_size, seqs
